# baseline (speedup 1.0000x reference)
import jax
import jax.numpy as jnp
from jax.experimental import pallas as pl

_M_TILE = 128
_N = 8192
_K = 32
_CHUNK = 2048
_BIG_IDX = 2**30


def _prep_body(x_ref, w_ref, c_ref, r_ref, wb_ref):
    x = x_ref[:]
    c_ref[:] = jnp.sqrt(jnp.sum(x * x, axis=0, keepdims=True))
    w = w_ref[:]
    ones = jnp.ones((1, _K), dtype=jnp.float32)
    r2 = jax.lax.dot_general(
        ones, w * w, (((1,), (1,)), ((), ())),
        preferred_element_type=jnp.float32,
        precision=jax.lax.Precision.HIGHEST,
    )
    r_ref[:] = jnp.sqrt(r2)
    wb_ref[:] = w.astype(jnp.bfloat16)


def _argmax_body(x_ref, wb_ref, c_ref, r_ref, o_ref):
    xs = (x_ref[:] / c_ref[:]).astype(jnp.bfloat16)
    beta = jax.lax.dot_general(
        xs, wb_ref[:], (((1,), (1,)), ((), ())),
        preferred_element_type=jnp.float32,
    )
    beta = beta / r_ref[:]

    acc_v = jnp.full((beta.shape[0], 1), -jnp.inf, dtype=jnp.float32)
    acc_i = jnp.zeros((beta.shape[0], 1), dtype=jnp.int32)
    for q in range(_N // _CHUNK):
        chunk = beta[:, q * _CHUNK:(q + 1) * _CHUNK]
        m_q = jnp.max(chunk, axis=1, keepdims=True)
        i_q = (jnp.argmax(chunk, axis=1).astype(jnp.int32)
               + q * _CHUNK)[:, None]
        take = m_q > acc_v
        acc_v = jnp.where(take, m_q.astype(jnp.bfloat16).astype(jnp.float32), acc_v)
        acc_i = jnp.where(take, i_q, acc_i)
    o_ref[:] = acc_i


def kernel(all_ts, W, clustering_flag):
    del clustering_flag
    m = all_ts.shape[0]
    x = jnp.reshape(all_ts, (m, _K))

    c, r, wb = pl.pallas_call(
        _prep_body,
        out_shape=(
            jax.ShapeDtypeStruct((1, _K), jnp.float32),
            jax.ShapeDtypeStruct((1, _N), jnp.float32),
            jax.ShapeDtypeStruct((_N, _K), jnp.bfloat16),
        ),
    )(x, W)

    n_star = pl.pallas_call(
        _argmax_body,
        grid=(m // _M_TILE,),
        in_specs=[
            pl.BlockSpec((_M_TILE, _K), lambda i: (i, 0)),
            pl.BlockSpec((_N, _K), lambda i: (0, 0)),
            pl.BlockSpec((1, _K), lambda i: (0, 0)),
            pl.BlockSpec((1, _N), lambda i: (0, 0)),
        ],
        out_specs=pl.BlockSpec((_M_TILE, 1), lambda i: (i, 0)),
        out_shape=jax.ShapeDtypeStruct((m, 1), jnp.int32),
    )(x, wb, c, r)

    return jnp.reshape(n_star, (m,))


# speedup vs baseline: 1.3002x; 1.3002x over previous
import jax
import jax.numpy as jnp
from jax.experimental import pallas as pl

_M_TILE = 256
_N = 8192
_K = 32
_CHUNK = 2048


def _prep_body(x_ref, w_ref, c_ref, r_ref, wb_ref):
    x = x_ref[:]
    c_ref[:] = jnp.sqrt(jnp.sum(x * x, axis=0, keepdims=True))
    w = w_ref[:]
    r_ref[:] = jnp.sqrt(jnp.sum(w * w, axis=1, keepdims=True))
    wb_ref[:] = w.astype(jnp.bfloat16)


def _argmax_body(x_ref, wb_ref, c_ref, r_ref, o_ref):
    xs = (x_ref[:] / c_ref[:]).astype(jnp.bfloat16)
    beta = jax.lax.dot_general(
        wb_ref[:], xs, (((1,), (1,)), ((), ())),
        preferred_element_type=jnp.float32,
    )
    beta = beta / r_ref[:]

    acc_v = jnp.full((1, beta.shape[1]), -jnp.inf, dtype=jnp.float32)
    acc_i = jnp.zeros((1, beta.shape[1]), dtype=jnp.int32)
    for q in range(_N // _CHUNK):
        chunk = beta[q * _CHUNK:(q + 1) * _CHUNK, :]
        m_q = jnp.max(chunk, axis=0, keepdims=True)
        i_q = (jnp.argmax(chunk, axis=0).astype(jnp.int32)
               + q * _CHUNK)[None, :]
        take = m_q > acc_v
        acc_v = jnp.where(take, m_q.astype(jnp.bfloat16).astype(jnp.float32), acc_v)
        acc_i = jnp.where(take, i_q, acc_i)
    o_ref[:] = acc_i


def kernel(all_ts, W, clustering_flag):
    del clustering_flag
    m = all_ts.shape[0]
    x = jnp.reshape(all_ts, (m, _K))

    c, r, wb = pl.pallas_call(
        _prep_body,
        out_shape=(
            jax.ShapeDtypeStruct((1, _K), jnp.float32),
            jax.ShapeDtypeStruct((_N, 1), jnp.float32),
            jax.ShapeDtypeStruct((_N, _K), jnp.bfloat16),
        ),
    )(x, W)

    n_star = pl.pallas_call(
        _argmax_body,
        grid=(m // _M_TILE,),
        in_specs=[
            pl.BlockSpec((_M_TILE, _K), lambda i: (i, 0)),
            pl.BlockSpec((_N, _K), lambda i: (0, 0)),
            pl.BlockSpec((1, _K), lambda i: (0, 0)),
            pl.BlockSpec((_N, 1), lambda i: (0, 0)),
        ],
        out_specs=pl.BlockSpec((1, _M_TILE), lambda i: (0, i)),
        out_shape=jax.ShapeDtypeStruct((1, m), jnp.int32),
    )(x, wb, c, r)

    return jnp.reshape(n_star, (m,))


# speedup vs baseline: 1.4641x; 1.1260x over previous
import jax
import jax.numpy as jnp
from jax.experimental import pallas as pl

_M_TILE = 512
_N = 8192
_K = 32
_CHUNK = 2048


def _prep_body(x_ref, w_ref, c_ref, r_ref, wb_ref):
    x = x_ref[:]
    c_ref[:] = jnp.sqrt(jnp.sum(x * x, axis=0, keepdims=True))
    w = w_ref[:]
    r_ref[:] = jnp.sqrt(jnp.sum(w * w, axis=1, keepdims=True))
    wb_ref[:] = w.astype(jnp.bfloat16)


def _argmax_body(x_ref, wb_ref, c_ref, r_ref, o_ref):
    xs = (x_ref[:] / c_ref[:]).astype(jnp.bfloat16)
    beta = jax.lax.dot_general(
        wb_ref[:], xs, (((1,), (1,)), ((), ())),
        preferred_element_type=jnp.float32,
    )
    beta = beta / r_ref[:]

    acc_v = jnp.full((1, beta.shape[1]), -jnp.inf, dtype=jnp.float32)
    acc_i = jnp.zeros((1, beta.shape[1]), dtype=jnp.int32)
    for q in range(_N // _CHUNK):
        chunk = beta[q * _CHUNK:(q + 1) * _CHUNK, :]
        m_q = jnp.max(chunk, axis=0, keepdims=True)
        i_q = (jnp.argmax(chunk, axis=0).astype(jnp.int32)
               + q * _CHUNK)[None, :]
        take = m_q > acc_v
        acc_v = jnp.where(take, m_q.astype(jnp.bfloat16).astype(jnp.float32), acc_v)
        acc_i = jnp.where(take, i_q, acc_i)
    o_ref[:] = acc_i


def kernel(all_ts, W, clustering_flag):
    del clustering_flag
    m = all_ts.shape[0]
    x = jnp.reshape(all_ts, (m, _K))

    c, r, wb = pl.pallas_call(
        _prep_body,
        out_shape=(
            jax.ShapeDtypeStruct((1, _K), jnp.float32),
            jax.ShapeDtypeStruct((_N, 1), jnp.float32),
            jax.ShapeDtypeStruct((_N, _K), jnp.bfloat16),
        ),
    )(x, W)

    n_star = pl.pallas_call(
        _argmax_body,
        grid=(m // _M_TILE,),
        in_specs=[
            pl.BlockSpec((_M_TILE, _K), lambda i: (i, 0)),
            pl.BlockSpec((_N, _K), lambda i: (0, 0)),
            pl.BlockSpec((1, _K), lambda i: (0, 0)),
            pl.BlockSpec((_N, 1), lambda i: (0, 0)),
        ],
        out_specs=pl.BlockSpec((1, _M_TILE), lambda i: (0, i)),
        out_shape=jax.ShapeDtypeStruct((1, m), jnp.int32),
    )(x, wb, c, r)

    return jnp.reshape(n_star, (m,))


# speedup vs baseline: 1.5693x; 1.0719x over previous
import jax
import jax.numpy as jnp
from jax.experimental import pallas as pl

_M_TILE = 1024
_N = 8192
_K = 32
_CHUNK = 2048


def _prep_body(x_ref, w_ref, c_ref, r_ref, wb_ref):
    x = x_ref[:]
    c_ref[:] = jnp.sqrt(jnp.sum(x * x, axis=0, keepdims=True))
    w = w_ref[:]
    r_ref[:] = jnp.sqrt(jnp.sum(w * w, axis=1, keepdims=True))
    wb_ref[:] = w.astype(jnp.bfloat16)


def _argmax_body(x_ref, wb_ref, c_ref, r_ref, o_ref):
    xs = (x_ref[:] / c_ref[:]).astype(jnp.bfloat16)
    beta = jax.lax.dot_general(
        wb_ref[:], xs, (((1,), (1,)), ((), ())),
        preferred_element_type=jnp.float32,
    )
    beta = beta / r_ref[:]

    acc_v = jnp.full((1, beta.shape[1]), -jnp.inf, dtype=jnp.float32)
    acc_i = jnp.zeros((1, beta.shape[1]), dtype=jnp.int32)
    for q in range(_N // _CHUNK):
        chunk = beta[q * _CHUNK:(q + 1) * _CHUNK, :]
        m_q = jnp.max(chunk, axis=0, keepdims=True)
        i_q = (jnp.argmax(chunk, axis=0).astype(jnp.int32)
               + q * _CHUNK)[None, :]
        take = m_q > acc_v
        acc_v = jnp.where(take, m_q.astype(jnp.bfloat16).astype(jnp.float32), acc_v)
        acc_i = jnp.where(take, i_q, acc_i)
    o_ref[:] = acc_i


def kernel(all_ts, W, clustering_flag):
    del clustering_flag
    m = all_ts.shape[0]
    x = jnp.reshape(all_ts, (m, _K))

    c, r, wb = pl.pallas_call(
        _prep_body,
        out_shape=(
            jax.ShapeDtypeStruct((1, _K), jnp.float32),
            jax.ShapeDtypeStruct((_N, 1), jnp.float32),
            jax.ShapeDtypeStruct((_N, _K), jnp.bfloat16),
        ),
    )(x, W)

    n_star = pl.pallas_call(
        _argmax_body,
        grid=(m // _M_TILE,),
        in_specs=[
            pl.BlockSpec((_M_TILE, _K), lambda i: (i, 0)),
            pl.BlockSpec((_N, _K), lambda i: (0, 0)),
            pl.BlockSpec((1, _K), lambda i: (0, 0)),
            pl.BlockSpec((_N, 1), lambda i: (0, 0)),
        ],
        out_specs=pl.BlockSpec((1, _M_TILE), lambda i: (0, i)),
        out_shape=jax.ShapeDtypeStruct((1, m), jnp.int32),
    )(x, wb, c, r)

    return jnp.reshape(n_star, (m,))


# speedup vs baseline: 1.6057x; 1.0232x over previous
import jax
import jax.numpy as jnp
from jax.experimental import pallas as pl

_M_TILE = 2048
_N = 8192
_K = 32
_CHUNK = 2048


def _prep_body(x_ref, w_ref, c_ref, r_ref, wb_ref):
    x = x_ref[:]
    c_ref[:] = jnp.sqrt(jnp.sum(x * x, axis=0, keepdims=True))
    w = w_ref[:]
    r_ref[:] = jnp.sqrt(jnp.sum(w * w, axis=1, keepdims=True))
    wb_ref[:] = w.astype(jnp.bfloat16)


def _argmax_body(x_ref, wb_ref, c_ref, r_ref, o_ref):
    xs = (x_ref[:] / c_ref[:]).astype(jnp.bfloat16)
    beta = jax.lax.dot_general(
        wb_ref[:], xs, (((1,), (1,)), ((), ())),
        preferred_element_type=jnp.float32,
    )
    beta = beta / r_ref[:]

    acc_v = jnp.full((1, beta.shape[1]), -jnp.inf, dtype=jnp.float32)
    acc_i = jnp.zeros((1, beta.shape[1]), dtype=jnp.int32)
    for q in range(_N // _CHUNK):
        chunk = beta[q * _CHUNK:(q + 1) * _CHUNK, :]
        m_q = jnp.max(chunk, axis=0, keepdims=True)
        i_q = (jnp.argmax(chunk, axis=0).astype(jnp.int32)
               + q * _CHUNK)[None, :]
        take = m_q > acc_v
        acc_v = jnp.where(take, m_q.astype(jnp.bfloat16).astype(jnp.float32), acc_v)
        acc_i = jnp.where(take, i_q, acc_i)
    o_ref[:] = acc_i


def kernel(all_ts, W, clustering_flag):
    del clustering_flag
    m = all_ts.shape[0]
    x = jnp.reshape(all_ts, (m, _K))

    c, r, wb = pl.pallas_call(
        _prep_body,
        out_shape=(
            jax.ShapeDtypeStruct((1, _K), jnp.float32),
            jax.ShapeDtypeStruct((_N, 1), jnp.float32),
            jax.ShapeDtypeStruct((_N, _K), jnp.bfloat16),
        ),
    )(x, W)

    n_star = pl.pallas_call(
        _argmax_body,
        grid=(m // _M_TILE,),
        in_specs=[
            pl.BlockSpec((_M_TILE, _K), lambda i: (i, 0)),
            pl.BlockSpec((_N, _K), lambda i: (0, 0)),
            pl.BlockSpec((1, _K), lambda i: (0, 0)),
            pl.BlockSpec((_N, 1), lambda i: (0, 0)),
        ],
        out_specs=pl.BlockSpec((1, _M_TILE), lambda i: (0, i)),
        out_shape=jax.ShapeDtypeStruct((1, m), jnp.int32),
    )(x, wb, c, r)

    return jnp.reshape(n_star, (m,))


# speedup vs baseline: 1.6204x; 1.0091x over previous
import jax
import jax.numpy as jnp
from jax.experimental import pallas as pl

_M_TILE = 4096
_N = 8192
_K = 32
_CHUNK = 2048


def _prep_body(x_ref, w_ref, c_ref, r_ref, wb_ref):
    x = x_ref[:]
    c_ref[:] = jnp.sqrt(jnp.sum(x * x, axis=0, keepdims=True))
    w = w_ref[:]
    r_ref[:] = jnp.sqrt(jnp.sum(w * w, axis=1, keepdims=True))
    wb_ref[:] = w.astype(jnp.bfloat16)


def _argmax_body(x_ref, wb_ref, c_ref, r_ref, o_ref):
    xs = (x_ref[:] / c_ref[:]).astype(jnp.bfloat16)
    beta = jax.lax.dot_general(
        wb_ref[:], xs, (((1,), (1,)), ((), ())),
        preferred_element_type=jnp.float32,
    )
    beta = beta / r_ref[:]

    acc_v = jnp.full((1, beta.shape[1]), -jnp.inf, dtype=jnp.float32)
    acc_i = jnp.zeros((1, beta.shape[1]), dtype=jnp.int32)
    for q in range(_N // _CHUNK):
        chunk = beta[q * _CHUNK:(q + 1) * _CHUNK, :]
        m_q = jnp.max(chunk, axis=0, keepdims=True)
        i_q = (jnp.argmax(chunk, axis=0).astype(jnp.int32)
               + q * _CHUNK)[None, :]
        take = m_q > acc_v
        acc_v = jnp.where(take, m_q.astype(jnp.bfloat16).astype(jnp.float32), acc_v)
        acc_i = jnp.where(take, i_q, acc_i)
    o_ref[:] = acc_i


def kernel(all_ts, W, clustering_flag):
    del clustering_flag
    m = all_ts.shape[0]
    x = jnp.reshape(all_ts, (m, _K))

    c, r, wb = pl.pallas_call(
        _prep_body,
        out_shape=(
            jax.ShapeDtypeStruct((1, _K), jnp.float32),
            jax.ShapeDtypeStruct((_N, 1), jnp.float32),
            jax.ShapeDtypeStruct((_N, _K), jnp.bfloat16),
        ),
    )(x, W)

    n_star = pl.pallas_call(
        _argmax_body,
        grid=(m // _M_TILE,),
        in_specs=[
            pl.BlockSpec((_M_TILE, _K), lambda i: (i, 0)),
            pl.BlockSpec((_N, _K), lambda i: (0, 0)),
            pl.BlockSpec((1, _K), lambda i: (0, 0)),
            pl.BlockSpec((_N, 1), lambda i: (0, 0)),
        ],
        out_specs=pl.BlockSpec((1, _M_TILE), lambda i: (0, i)),
        out_shape=jax.ShapeDtypeStruct((1, m), jnp.int32),
    )(x, wb, c, r)

    return jnp.reshape(n_star, (m,))
